# Initial kernel scaffold; baseline (speedup 1.0000x reference)
import jax
import jax.numpy as jnp
from jax.experimental import pallas as pl


def kernel(opcode_filters, method_indices):
    raise NotImplementedError("write your pallas kernel here")


# speedup vs baseline: 1.7490x; 1.7490x over previous
import functools

import jax
import jax.numpy as jnp
from jax import lax
from jax.experimental import pallas as pl
from jax.experimental.pallas import tpu as pltpu
from jax.experimental.pallas import tpu_sc as plsc

B = 16
S = 4096
F = 128
Q = 512
BLK = 512
NW = 32
QW = (B * Q) // NW
L = 16


def _cumsum_body(tri_ref, x_ref, mi_ref, o_ref, sidx_ref, eidx_ref, rcp_ref,
                 carry_ref):
    b = pl.program_id(0)
    j = pl.program_id(1)

    @pl.when(j == 0)
    def _():
        carry_ref[...] = jnp.zeros_like(carry_ref)
        s = jnp.clip(mi_ref[0, 0:1, :], 0, S - 1)
        e = jnp.clip(mi_ref[0, 1:2, :], 0, S - 1)
        sidx_ref[0] = s + b * S
        eidx_ref[0] = e + b * S
        rcp_ref[0] = 1.0 / (e - s + 1).astype(jnp.float32)

    x = x_ref[0]
    cs = jnp.dot(tri_ref[...], x, preferred_element_type=jnp.float32)
    cs = cs + carry_ref[...]
    o_ref[0] = cs
    carry_ref[...] = cs[BLK - 1:BLK, :]


def _stage1_tc(x, tri, mi_t):
    return pl.pallas_call(
        _cumsum_body,
        grid=(B, S // BLK),
        in_specs=[
            pl.BlockSpec((BLK, BLK), lambda b, j: (0, 0)),
            pl.BlockSpec((1, BLK, F), lambda b, j: (b, j, 0)),
            pl.BlockSpec((1, 2, Q), lambda b, j: (b, 0, 0)),
        ],
        out_specs=[
            pl.BlockSpec((1, BLK, F), lambda b, j: (b, j, 0)),
            pl.BlockSpec((1, 1, Q), lambda b, j: (b, 0, 0)),
            pl.BlockSpec((1, 1, Q), lambda b, j: (b, 0, 0)),
            pl.BlockSpec((1, 1, Q), lambda b, j: (b, 0, 0)),
        ],
        out_shape=[
            jax.ShapeDtypeStruct((B, S, F), jnp.float32),
            jax.ShapeDtypeStruct((B, 1, Q), jnp.int32),
            jax.ShapeDtypeStruct((B, 1, Q), jnp.int32),
            jax.ShapeDtypeStruct((B, 1, Q), jnp.float32),
        ],
        scratch_shapes=[pltpu.VMEM((1, F), jnp.float32)],
        compiler_params=pltpu.CompilerParams(
            dimension_semantics=("arbitrary", "arbitrary"),
        ),
    )(tri, x, mi_t)


def _gather_mean_body(cs_hbm, sidx_hbm, eidx_hbm, rcp_hbm, out_hbm,
                      sidx_v, eidx_v, rows_s, rows_e, rcp_v, out_v,
                      sem_s, sem_e):
    wid = lax.axis_index("s") * 2 + lax.axis_index("c")
    qbase = wid * QW

    pltpu.sync_copy(sidx_hbm.at[pl.ds(qbase, QW)], sidx_v)
    pltpu.sync_copy(eidx_hbm.at[pl.ds(qbase, QW)], eidx_v)
    pltpu.sync_copy(rcp_hbm.at[pl.ds(qbase, QW)], rcp_v)

    cp_s = pltpu.async_copy(cs_hbm.at[sidx_v], rows_s, sem_s)
    cp_e = pltpu.async_copy(cs_hbm.at[eidx_v], rows_e, sem_e)
    cp_s.wait()
    cp_e.wait()

    def _mean(i, _):
        rcpc = rcp_v[pl.ds(i * L, L)]
        for jq in range(L):
            q = i * L + jq
            r = rcpc[jq]
            for c in range(F // L):
                sl = pl.ds(c * L, L)
                out_v[q, sl] = (rows_e[q, sl] - rows_s[q, sl]) * r
        return 0

    lax.fori_loop(0, QW // L, _mean, 0)

    pltpu.sync_copy(out_v, out_hbm.at[pl.ds(qbase, QW)])


def _gather_mean_sc(cs_flat, sidx, eidx, rcp):
    mesh = plsc.VectorSubcoreMesh(core_axis_name="c", subcore_axis_name="s")
    return pl.kernel(
        _gather_mean_body,
        mesh=mesh,
        out_type=jax.ShapeDtypeStruct((B * Q, F), jnp.float32),
        scratch_types=[
            pltpu.VMEM((QW,), jnp.int32),
            pltpu.VMEM((QW,), jnp.int32),
            pltpu.VMEM((QW, F), jnp.float32),
            pltpu.VMEM((QW, F), jnp.float32),
            pltpu.VMEM((QW,), jnp.float32),
            pltpu.VMEM((QW, F), jnp.float32),
            pltpu.SemaphoreType.DMA,
            pltpu.SemaphoreType.DMA,
        ],
    )(cs_flat, sidx, eidx, rcp)


def kernel(opcode_filters, method_indices):
    tri = jnp.tril(jnp.ones((BLK, BLK), jnp.float32))
    mi_t = method_indices.transpose(0, 2, 1)
    cs, sidx, eidx, rcp = _stage1_tc(opcode_filters, tri, mi_t)
    out = _gather_mean_sc(
        cs.reshape(B * S, F),
        sidx.reshape(B * Q),
        eidx.reshape(B * Q),
        rcp.reshape(B * Q),
    )
    return out.reshape(B, Q, F)
